# baseline (speedup 1.0000x reference)
import functools

import jax
import jax.numpy as jnp
import numpy as np
from jax.experimental import pallas as pl

_NUM_BINS = 12
_ANGLE_PER_BIN = 2.0 * np.pi / _NUM_BINS
_TWO_PI = 2.0 * np.pi


def _rpn_kernel(x_ref, xyz_ref, w1_ref, b1_ref, w2_ref, b2_ref,
                wobj_ref, bobj_ref, wbox_ref, bbox_ref,
                whcls_ref, bhcls_ref, whd_ref, bhd_ref,
                obj_ref, box_ref, ang_ref):
    x = x_ref[...]
    h = jnp.maximum(
        jax.lax.dot_general(w1_ref[...], x, (((0,), (1,)), ((), ())),
                            preferred_element_type=jnp.float32)
        + b1_ref[...], 0.0)
    h = jnp.maximum(
        jax.lax.dot_general(w2_ref[...], h, (((0,), (0,)), ((), ())),
                            preferred_element_type=jnp.float32)
        + b2_ref[...], 0.0)

    def head(w_ref, b_ref):
        return (jax.lax.dot_general(w_ref[...], h, (((0,), (0,)), ((), ())),
                                    preferred_element_type=jnp.float32)
                + b_ref[...])

    obj_ref[...] = jax.nn.sigmoid(head(wobj_ref, bobj_ref))

    box = head(wbox_ref, bbox_ref)
    box_t = box.T
    xyz = xyz_ref[...]
    mins = xyz - box_t[:, 0:3]
    maxs = xyz + box_t[:, 3:6]
    box_ref[...] = jnp.concatenate([mins, maxs], axis=1)

    hcls = head(whcls_ref, bhcls_ref)
    hd = head(whd_ref, bhd_ref)
    mx = jnp.max(hcls, axis=0, keepdims=True)
    iota = jax.lax.broadcasted_iota(jnp.int32, hcls.shape, 0)
    idx = jnp.min(jnp.where(hcls == mx, iota, _NUM_BINS),
                  axis=0, keepdims=True)
    delta = jnp.sum(jnp.where(iota == idx, hd, 0.0), axis=0, keepdims=True)
    ang_ref[...] = jnp.mod(idx.astype(jnp.float32) * _ANGLE_PER_BIN + delta,
                           _TWO_PI)


@functools.partial(jax.jit, static_argnames=())
def kernel(voted_xyz, voted_features, W1, b1, W2, b2, W_obj, b_obj,
           W_box, b_box, W_hcls, b_hcls, W_hd, b_hd):
    B, N, C = voted_features.shape
    H = W1.shape[1]
    M = B * N
    T = 1024
    grid = (M // T,)

    x = voted_features.reshape(M, C)
    xyz = voted_xyz.reshape(M, 3)

    def const(shape):
        ndim = len(shape)
        return pl.BlockSpec(shape, lambda i: (0,) * ndim)

    obj, boxes, ang = pl.pallas_call(
        _rpn_kernel,
        grid=grid,
        in_specs=[
            pl.BlockSpec((T, C), lambda i: (i, 0)),
            pl.BlockSpec((T, 3), lambda i: (i, 0)),
            const((C, H)), const((H, 1)),
            const((H, H)), const((H, 1)),
            const((H, 1)), const((1, 1)),
            const((H, 6)), const((6, 1)),
            const((H, _NUM_BINS)), const((_NUM_BINS, 1)),
            const((H, _NUM_BINS)), const((_NUM_BINS, 1)),
        ],
        out_specs=[
            pl.BlockSpec((1, T), lambda i: (0, i)),
            pl.BlockSpec((T, 6), lambda i: (i, 0)),
            pl.BlockSpec((1, T), lambda i: (0, i)),
        ],
        out_shape=[
            jax.ShapeDtypeStruct((1, M), jnp.float32),
            jax.ShapeDtypeStruct((M, 6), jnp.float32),
            jax.ShapeDtypeStruct((1, M), jnp.float32),
        ],
    )(x, xyz, W1, b1.reshape(H, 1), W2, b2.reshape(H, 1),
      W_obj, b_obj.reshape(1, 1), W_box, b_box.reshape(6, 1),
      W_hcls, b_hcls.reshape(_NUM_BINS, 1), W_hd, b_hd.reshape(_NUM_BINS, 1))

    return (obj.reshape(B, N), boxes.reshape(B, N, 6), ang.reshape(B, N))


# speedup vs baseline: 1.2737x; 1.2737x over previous
import functools

import jax
import jax.numpy as jnp
import numpy as np
from jax.experimental import pallas as pl

_NUM_BINS = 12
_ANGLE_PER_BIN = 2.0 * np.pi / _NUM_BINS
_TWO_PI = 2.0 * np.pi


def _rpn_kernel(x_ref, xyzt_ref, w1_ref, b1_ref, w2_ref, b2_ref,
                wh_ref, bh_ref, obj_ref, box_ref, ang_ref):
    x = x_ref[...]
    h = jnp.maximum(
        jax.lax.dot_general(w1_ref[...], x, (((0,), (1,)), ((), ())),
                            preferred_element_type=jnp.float32)
        + b1_ref[...], 0.0)
    h = jnp.maximum(
        jax.lax.dot_general(w2_ref[...], h, (((0,), (0,)), ((), ())),
                            preferred_element_type=jnp.float32)
        + b2_ref[...], 0.0)
    o = (jax.lax.dot_general(wh_ref[...], h, (((0,), (0,)), ((), ())),
                             preferred_element_type=jnp.float32)
         + bh_ref[...])

    obj_ref[...] = jax.nn.sigmoid(o[0:1, :])
    xyz = xyzt_ref[...]
    box_ref[...] = jnp.concatenate(
        [xyz - o[1:4, :], xyz + o[4:7, :]], axis=0)

    hcls = o[7:7 + _NUM_BINS, :]
    hd = o[7 + _NUM_BINS:7 + 2 * _NUM_BINS, :]
    mx = jnp.max(hcls, axis=0, keepdims=True)
    iota = jax.lax.broadcasted_iota(jnp.int32, hcls.shape, 0)
    idx = jnp.min(jnp.where(hcls == mx, iota, _NUM_BINS),
                  axis=0, keepdims=True)
    delta = jnp.sum(jnp.where(iota == idx, hd, 0.0), axis=0, keepdims=True)
    ang_ref[...] = jnp.mod(idx.astype(jnp.float32) * _ANGLE_PER_BIN + delta,
                           _TWO_PI)


@functools.partial(jax.jit, static_argnames=())
def kernel(voted_xyz, voted_features, W1, b1, W2, b2, W_obj, b_obj,
           W_box, b_box, W_hcls, b_hcls, W_hd, b_hd):
    B, N, C = voted_features.shape
    H = W1.shape[1]
    M = B * N
    T = 1024
    grid = (M // T,)

    x = voted_features.reshape(M, C)
    xyz_t = voted_xyz.reshape(M, 3).T
    wh = jnp.concatenate(
        [W_obj, W_box, W_hcls, W_hd,
         jnp.zeros((H, 1), dtype=W_obj.dtype)], axis=1)
    bh = jnp.concatenate(
        [b_obj, b_box, b_hcls, b_hd,
         jnp.zeros((1,), dtype=b_obj.dtype)], axis=0)

    obj, boxes_t, ang = pl.pallas_call(
        _rpn_kernel,
        grid=grid,
        in_specs=[
            pl.BlockSpec((T, C), lambda i: (i, 0)),
            pl.BlockSpec((3, T), lambda i: (0, i)),
            pl.BlockSpec((C, H), lambda i: (0, 0)),
            pl.BlockSpec((H, 1), lambda i: (0, 0)),
            pl.BlockSpec((H, H), lambda i: (0, 0)),
            pl.BlockSpec((H, 1), lambda i: (0, 0)),
            pl.BlockSpec((H, 32), lambda i: (0, 0)),
            pl.BlockSpec((32, 1), lambda i: (0, 0)),
        ],
        out_specs=[
            pl.BlockSpec((1, T), lambda i: (0, i)),
            pl.BlockSpec((6, T), lambda i: (0, i)),
            pl.BlockSpec((1, T), lambda i: (0, i)),
        ],
        out_shape=[
            jax.ShapeDtypeStruct((1, M), jnp.float32),
            jax.ShapeDtypeStruct((6, M), jnp.float32),
            jax.ShapeDtypeStruct((1, M), jnp.float32),
        ],
    )(x, xyz_t, W1, b1.reshape(H, 1), W2, b2.reshape(H, 1),
      wh, bh.reshape(32, 1))

    return (obj.reshape(B, N), boxes_t.T.reshape(B, N, 6), ang.reshape(B, N))


# speedup vs baseline: 1.4874x; 1.1678x over previous
import functools

import jax
import jax.numpy as jnp
import numpy as np
from jax.experimental import pallas as pl

_NUM_BINS = 12
_ANGLE_PER_BIN = 2.0 * np.pi / _NUM_BINS
_TWO_PI = 2.0 * np.pi


def _rpn_kernel(x_ref, w1_ref, b1_ref, w2_ref, b2_ref,
                wh_ref, bh_ref, out_ref):
    x = x_ref[...]
    h = jnp.maximum(
        jax.lax.dot_general(w1_ref[...], x, (((0,), (1,)), ((), ())),
                            preferred_element_type=jnp.float32)
        + b1_ref[...], 0.0)
    h = jnp.maximum(
        jax.lax.dot_general(w2_ref[...], h, (((0,), (0,)), ((), ())),
                            preferred_element_type=jnp.float32)
        + b2_ref[...], 0.0)
    o = (jax.lax.dot_general(wh_ref[...], h, (((0,), (0,)), ((), ())),
                             preferred_element_type=jnp.float32)
         + bh_ref[...])

    obj = jax.nn.sigmoid(o[0:1, :])

    hcls = o[7:7 + _NUM_BINS, :]
    hd = o[7 + _NUM_BINS:7 + 2 * _NUM_BINS, :]
    mx = jnp.max(hcls, axis=0, keepdims=True)
    iota = jax.lax.broadcasted_iota(jnp.int32, hcls.shape, 0)
    idx = jnp.min(jnp.where(hcls == mx, iota, _NUM_BINS),
                  axis=0, keepdims=True)
    delta = jnp.sum(jnp.where(iota == idx, hd, 0.0), axis=0, keepdims=True)
    ang = jnp.mod(idx.astype(jnp.float32) * _ANGLE_PER_BIN + delta, _TWO_PI)

    out_ref[...] = jnp.concatenate([obj, ang, o[1:7, :]], axis=0)


@functools.partial(jax.jit, static_argnames=())
def kernel(voted_xyz, voted_features, W1, b1, W2, b2, W_obj, b_obj,
           W_box, b_box, W_hcls, b_hcls, W_hd, b_hd):
    B, N, C = voted_features.shape
    H = W1.shape[1]
    M = B * N
    T = 1024
    grid = (M // T,)

    x = voted_features.reshape(M, C)
    wh = jnp.concatenate(
        [W_obj, W_box, W_hcls, W_hd,
         jnp.zeros((H, 1), dtype=W_obj.dtype)], axis=1)
    bh = jnp.concatenate(
        [b_obj, b_box, b_hcls, b_hd,
         jnp.zeros((1,), dtype=b_obj.dtype)], axis=0)

    out = pl.pallas_call(
        _rpn_kernel,
        grid=grid,
        in_specs=[
            pl.BlockSpec((T, C), lambda i: (i, 0)),
            pl.BlockSpec((C, H), lambda i: (0, 0)),
            pl.BlockSpec((H, 1), lambda i: (0, 0)),
            pl.BlockSpec((H, H), lambda i: (0, 0)),
            pl.BlockSpec((H, 1), lambda i: (0, 0)),
            pl.BlockSpec((H, 32), lambda i: (0, 0)),
            pl.BlockSpec((32, 1), lambda i: (0, 0)),
        ],
        out_specs=pl.BlockSpec((8, T), lambda i: (0, i)),
        out_shape=jax.ShapeDtypeStruct((8, M), jnp.float32),
    )(x, W1, b1.reshape(H, 1), W2, b2.reshape(H, 1), wh, bh.reshape(32, 1))

    obj = out[0].reshape(B, N)
    ang = out[1].reshape(B, N)
    d = out[2:8].T
    xyz = voted_xyz.reshape(M, 3)
    boxes = jnp.concatenate([xyz - d[:, 0:3], xyz + d[:, 3:6]],
                            axis=-1).reshape(B, N, 6)
    return (obj, boxes, ang)


# speedup vs baseline: 1.6783x; 1.1283x over previous
import functools

import jax
import jax.numpy as jnp
import numpy as np
from jax.experimental import pallas as pl

_NUM_BINS = 12
_ANGLE_PER_BIN = 2.0 * np.pi / _NUM_BINS
_TWO_PI = 2.0 * np.pi


def _rpn_kernel(x_ref, w1_ref, b1_ref, w2_ref, b2_ref,
                wh_ref, bh_ref, out_ref):
    x = x_ref[...]
    h = jnp.maximum(
        jax.lax.dot_general(w1_ref[...], x, (((0,), (1,)), ((), ())),
                            preferred_element_type=jnp.float32)
        + b1_ref[...], 0.0)
    h = jnp.maximum(
        jax.lax.dot_general(w2_ref[...], h, (((0,), (0,)), ((), ())),
                            preferred_element_type=jnp.float32)
        + b2_ref[...], 0.0)
    o = (jax.lax.dot_general(wh_ref[...], h, (((0,), (0,)), ((), ())),
                             preferred_element_type=jnp.float32)
         + bh_ref[...])

    obj = jax.nn.sigmoid(o[0:1, :])

    hcls = o[7:7 + _NUM_BINS, :]
    hd = o[7 + _NUM_BINS:7 + 2 * _NUM_BINS, :]
    mx = jnp.max(hcls, axis=0, keepdims=True)
    iota = jax.lax.broadcasted_iota(jnp.int32, hcls.shape, 0)
    idx = jnp.min(jnp.where(hcls == mx, iota, _NUM_BINS),
                  axis=0, keepdims=True)
    delta = jnp.sum(jnp.where(iota == idx, hd, 0.0), axis=0, keepdims=True)
    ang = jnp.mod(idx.astype(jnp.float32) * _ANGLE_PER_BIN + delta, _TWO_PI)

    out_ref[...] = jnp.concatenate([obj, ang, o[1:7, :]], axis=0)


@functools.partial(jax.jit, static_argnames=())
def kernel(voted_xyz, voted_features, W1, b1, W2, b2, W_obj, b_obj,
           W_box, b_box, W_hcls, b_hcls, W_hd, b_hd):
    B, N, C = voted_features.shape
    H = W1.shape[1]
    M = B * N
    T = 2048
    grid = (M // T,)

    x = voted_features.reshape(M, C)
    wh = jnp.concatenate(
        [W_obj, W_box, W_hcls, W_hd,
         jnp.zeros((H, 1), dtype=W_obj.dtype)], axis=1)
    bh = jnp.concatenate(
        [b_obj, b_box, b_hcls, b_hd,
         jnp.zeros((1,), dtype=b_obj.dtype)], axis=0)

    out = pl.pallas_call(
        _rpn_kernel,
        grid=grid,
        in_specs=[
            pl.BlockSpec((T, C), lambda i: (i, 0)),
            pl.BlockSpec((C, H), lambda i: (0, 0)),
            pl.BlockSpec((H, 1), lambda i: (0, 0)),
            pl.BlockSpec((H, H), lambda i: (0, 0)),
            pl.BlockSpec((H, 1), lambda i: (0, 0)),
            pl.BlockSpec((H, 32), lambda i: (0, 0)),
            pl.BlockSpec((32, 1), lambda i: (0, 0)),
        ],
        out_specs=pl.BlockSpec((8, T), lambda i: (0, i)),
        out_shape=jax.ShapeDtypeStruct((8, M), jnp.float32),
    )(x, W1, b1.reshape(H, 1), W2, b2.reshape(H, 1), wh, bh.reshape(32, 1))

    obj = out[0].reshape(B, N)
    ang = out[1].reshape(B, N)
    d = out[2:8].T
    xyz = voted_xyz.reshape(M, 3)
    boxes = jnp.concatenate([xyz - d[:, 0:3], xyz + d[:, 3:6]],
                            axis=-1).reshape(B, N, 6)
    return (obj, boxes, ang)


# speedup vs baseline: 1.7333x; 1.0328x over previous
import functools

import jax
import jax.numpy as jnp
import numpy as np
from jax.experimental import pallas as pl

_NUM_BINS = 12
_ANGLE_PER_BIN = 2.0 * np.pi / _NUM_BINS
_TWO_PI = 2.0 * np.pi


def _rpn_kernel(x_ref, w1_ref, b1_ref, w2_ref, b2_ref,
                wh_ref, bh_ref, out_ref):
    x = x_ref[...]
    h = jnp.maximum(
        jax.lax.dot_general(w1_ref[...], x, (((0,), (1,)), ((), ())),
                            preferred_element_type=jnp.float32)
        + b1_ref[...], 0.0)
    h = jnp.maximum(
        jax.lax.dot_general(w2_ref[...], h, (((0,), (0,)), ((), ())),
                            preferred_element_type=jnp.float32)
        + b2_ref[...], 0.0)
    o = (jax.lax.dot_general(wh_ref[...], h, (((0,), (0,)), ((), ())),
                             preferred_element_type=jnp.float32)
         + bh_ref[...])

    obj = jax.nn.sigmoid(o[0:1, :])

    hcls = o[7:7 + _NUM_BINS, :]
    hd = o[7 + _NUM_BINS:7 + 2 * _NUM_BINS, :]
    mx = jnp.max(hcls, axis=0, keepdims=True)
    iota = jax.lax.broadcasted_iota(jnp.int32, hcls.shape, 0)
    idx = jnp.min(jnp.where(hcls == mx, iota, _NUM_BINS),
                  axis=0, keepdims=True)
    delta = jnp.sum(jnp.where(iota == idx, hd, 0.0), axis=0, keepdims=True)
    ang = jnp.mod(idx.astype(jnp.float32) * _ANGLE_PER_BIN + delta, _TWO_PI)

    out_ref[...] = jnp.concatenate([obj, ang, o[1:7, :]], axis=0)


@functools.partial(jax.jit, static_argnames=())
def kernel(voted_xyz, voted_features, W1, b1, W2, b2, W_obj, b_obj,
           W_box, b_box, W_hcls, b_hcls, W_hd, b_hd):
    B, N, C = voted_features.shape
    H = W1.shape[1]
    M = B * N
    T = 4096
    grid = (M // T,)

    x = voted_features.reshape(M, C)
    wh = jnp.concatenate(
        [W_obj, W_box, W_hcls, W_hd,
         jnp.zeros((H, 1), dtype=W_obj.dtype)], axis=1)
    bh = jnp.concatenate(
        [b_obj, b_box, b_hcls, b_hd,
         jnp.zeros((1,), dtype=b_obj.dtype)], axis=0)

    out = pl.pallas_call(
        _rpn_kernel,
        grid=grid,
        in_specs=[
            pl.BlockSpec((T, C), lambda i: (i, 0)),
            pl.BlockSpec((C, H), lambda i: (0, 0)),
            pl.BlockSpec((H, 1), lambda i: (0, 0)),
            pl.BlockSpec((H, H), lambda i: (0, 0)),
            pl.BlockSpec((H, 1), lambda i: (0, 0)),
            pl.BlockSpec((H, 32), lambda i: (0, 0)),
            pl.BlockSpec((32, 1), lambda i: (0, 0)),
        ],
        out_specs=pl.BlockSpec((8, T), lambda i: (0, i)),
        out_shape=jax.ShapeDtypeStruct((8, M), jnp.float32),
    )(x, W1, b1.reshape(H, 1), W2, b2.reshape(H, 1), wh, bh.reshape(32, 1))

    obj = out[0].reshape(B, N)
    ang = out[1].reshape(B, N)
    d = out[2:8].T
    xyz = voted_xyz.reshape(M, 3)
    boxes = jnp.concatenate([xyz - d[:, 0:3], xyz + d[:, 3:6]],
                            axis=-1).reshape(B, N, 6)
    return (obj, boxes, ang)
